# Initial kernel scaffold; baseline (speedup 1.0000x reference)
import jax
import jax.numpy as jnp
from jax.experimental import pallas as pl


def kernel(x, edge_index, edge_weight, graph_indicator, W1, b1, W2, b2, centroids, mlp_W, mlp_b):
    raise NotImplementedError("write your pallas kernel here")


# speedup vs baseline: 3.7717x; 3.7717x over previous
import functools

import jax
import jax.numpy as jnp
from jax import lax
from jax.experimental import pallas as pl
from jax.experimental.pallas import tpu as pltpu
from jax.experimental.pallas import tpu_sc as plsc

N = 10000
E = 320000
D = 128
K = 64
NCLS = 10
MINN = 1e-15
EPS = 1e-7
MAXNORM = 1.0 - 1e-3

NC = 2
NS = 16
NW = NC * NS
EPW = E // NW
CH = 80
NCHUNK = EPW // CH
NPAD = 10240
RPT = NPAD // NS

BN = 1000


def _artanh(x):
    x = jnp.clip(x, -1.0 + EPS, 1.0 - EPS)
    return 0.5 * jnp.log((1.0 + x) / (1.0 - x))


def _rownorm(x):
    return jnp.maximum(jnp.sqrt(jnp.sum(x * x, axis=-1, keepdims=True)), MINN)


def _proj(x):
    norm = _rownorm(x)
    return jnp.where(norm > MAXNORM, x / norm * MAXNORM, x)


def _expmap0(u):
    n = _rownorm(u)
    return jnp.tanh(n) * u / n


def _logmap0(p):
    n = _rownorm(p)
    return _artanh(n) * p / n


def _mobius_add(x, y):
    x2 = jnp.sum(x * x, axis=-1, keepdims=True)
    y2 = jnp.sum(y * y, axis=-1, keepdims=True)
    xy = jnp.sum(x * y, axis=-1, keepdims=True)
    num = (1.0 + 2.0 * xy + y2) * x + (1.0 - x2) * y
    den = 1.0 + 2.0 * xy + x2 * y2
    return num / jnp.maximum(den, MINN)


def _mobius_matvec(x, W):
    xn = _rownorm(x)
    mx = jnp.dot(x, W, preferred_element_type=jnp.float32)
    mxn = _rownorm(mx)
    res = jnp.tanh(mxn / xn * _artanh(xn)) * mx / mxn
    cond = jnp.sum(jnp.abs(mx), axis=-1, keepdims=True) == 0
    return jnp.where(cond, jnp.zeros_like(res), res)


def _pre_agg(xh, W, b):
    mv = _proj(_mobius_matvec(xh, W))
    hb = _proj(_expmap0(b))
    h = _proj(_mobius_add(mv, hb))
    return _logmap0(h)


def _post_agg(support):
    h = _proj(_expmap0(support))
    ht = jax.nn.relu(_logmap0(h))
    return _proj(_expmap0(ht))


def _stage_a_body(x_ref, w_ref, b_ref, xhyp_ref, xt_ref):
    x = x_ref[...]
    xh = _proj(_expmap0(x))
    xhyp_ref[...] = xh
    xt_ref[...] = _pre_agg(xh, w_ref[...], b_ref[...])


def _stage_c_body(p0_ref, p1_ref, w_ref, b_ref, xt_ref):
    out1 = _post_agg(p0_ref[...] + p1_ref[...])
    xt_ref[...] = _pre_agg(out1, w_ref[...], b_ref[...])


def _stage_e_body(p0_ref, p1_ref, xh_ref, c_ref, acc_ref):
    i = pl.program_id(0)
    out2 = _post_agg(p0_ref[...] + p1_ref[...])
    out = out2 + xh_ref[...]
    cen = c_ref[...]
    x2 = jnp.sum(out * out, axis=-1, keepdims=True)
    y2 = jnp.sum(cen * cen, axis=-1)
    xy = lax.dot_general(out, cen, (((1,), (1,)), ((), ())),
                         preferred_element_type=jnp.float32)
    a = 1.0 - 2.0 * xy + y2[None, :]
    b = 1.0 - x2
    dd = 1.0 - 2.0 * xy + x2 * y2[None, :]
    sq = (a * a * x2 - 2.0 * a * b * xy + b * b * y2[None, :]) / jnp.maximum(dd * dd, MINN)
    nrm = jnp.sqrt(jnp.maximum(sq, 0.0))
    dist = 2.0 * _artanh(nrm)

    @pl.when(i == 0)
    def _():
        acc_ref[...] = jnp.zeros_like(acc_ref)

    acc_ref[...] += jnp.sum(dist, axis=0, keepdims=True)


_row_spec = pl.BlockSpec((BN, D), lambda i: (i, 0))
_whole = lambda shape: pl.BlockSpec(shape, lambda i: tuple(0 for _ in shape))


def _stage_a(x, W1, b1):
    return pl.pallas_call(
        _stage_a_body,
        grid=(N // BN,),
        in_specs=[_row_spec, _whole((D, D)), _whole((1, D))],
        out_specs=[_row_spec, _row_spec],
        out_shape=[jax.ShapeDtypeStruct((N, D), jnp.float32),
                   jax.ShapeDtypeStruct((N, D), jnp.float32)],
    )(x, W1, b1)


def _stage_c(p0, p1, W2, b2):
    return pl.pallas_call(
        _stage_c_body,
        grid=(N // BN,),
        in_specs=[_row_spec, _row_spec, _whole((D, D)), _whole((1, D))],
        out_specs=_row_spec,
        out_shape=jax.ShapeDtypeStruct((N, D), jnp.float32),
    )(p0, p1, W2, b2)


def _stage_e(p0, p1, xhyp, centroids):
    return pl.pallas_call(
        _stage_e_body,
        grid=(N // BN,),
        in_specs=[_row_spec, _row_spec, _row_spec, _whole((K, D))],
        out_specs=_whole((1, K)),
        out_shape=jax.ShapeDtypeStruct((1, K), jnp.float32),
    )(p0, p1, xhyp, centroids)


def _spmm_body(xt_hbm, src_hbm, dst_hbm, w_hbm, zeros_hbm, out_hbm,
               src_v, dst_v, w_s, rows_v, acc_sh, sem):
    cid = lax.axis_index("c")
    sid = lax.axis_index("s")
    wid = cid * NS + sid
    base = wid * EPW

    pltpu.sync_copy(zeros_hbm, acc_sh.at[pl.ds(sid * RPT, RPT)])
    plsc.subcore_barrier()

    @pl.loop(0, NCHUNK)
    def _(k):
        off = base + k * CH
        pltpu.sync_copy(src_hbm.at[pl.ds(off, CH)], src_v)
        pltpu.sync_copy(dst_hbm.at[pl.ds(off, CH)], dst_v)
        pltpu.sync_copy(w_hbm.at[pl.ds(off, CH)], w_s)
        pltpu.async_copy(xt_hbm.at[src_v], rows_v, sem).wait()

        @pl.loop(0, CH, step=16)
        def _(g):
            wv = w_s[pl.ds(g, 16)]
            for j in range(16):
                w = wv[j]
                for q in range(D // 16):
                    sl = (g + j, pl.ds(q * 16, 16))
                    rows_v[sl] = rows_v[sl] * w

        pltpu.sync_copy(rows_v, acc_sh.at[dst_v], add=True)

    plsc.subcore_barrier()
    pltpu.sync_copy(acc_sh.at[pl.ds(sid * RPT, RPT)],
                    out_hbm.at[cid, pl.ds(sid * RPT, RPT)])


@functools.cache
def _spmm_kernel():
    return pl.kernel(
        _spmm_body,
        out_type=jax.ShapeDtypeStruct((NC, NPAD, D), jnp.float32),
        mesh=plsc.VectorSubcoreMesh(core_axis_name="c", subcore_axis_name="s",
                                    num_cores=NC, num_subcores=NS),
        scratch_types=[
            pltpu.VMEM((CH,), jnp.int32),
            pltpu.VMEM((CH,), jnp.int32),
            pltpu.VMEM((CH,), jnp.float32),
            pltpu.VMEM((CH, D), jnp.float32),
            pltpu.VMEM_SHARED((NPAD, D), jnp.float32),
            pltpu.SemaphoreType.DMA,
        ],
    )


def _spmm(*args):
    return _spmm_kernel()(*args)


def kernel(x, edge_index, edge_weight, graph_indicator, W1, b1, W2, b2,
           centroids, mlp_W, mlp_b):
    del graph_indicator
    src = edge_index[0]
    dst = edge_index[1]
    b1r = b1.reshape(1, D)
    b2r = b2.reshape(1, D)
    zeros = jnp.zeros((RPT, D), jnp.float32)

    x_hyp, xt1 = _stage_a(x, W1, b1r)
    p1 = _spmm(xt1, src, dst, edge_weight, zeros)
    xt2 = _stage_c(p1[0], p1[1], W2, b2r)
    p2 = _spmm(xt2, src, dst, edge_weight, zeros)
    dist_sum = _stage_e(p2[0], p2[1], x_hyp, centroids)

    readout = dist_sum / jnp.float32(N)
    rn = jnp.maximum(jnp.sqrt(jnp.sum(readout * readout, axis=-1, keepdims=True)), MINN)
    h = _artanh(rn) * readout / rn
    logits = jax.nn.softmax(h @ mlp_W + mlp_b[None, :], axis=0)[None, :, :]
    return (logits, h)


# speedup vs baseline: 8.4840x; 2.2494x over previous
import functools

import jax
import jax.numpy as jnp
from jax import lax
from jax.experimental import pallas as pl
from jax.experimental.pallas import tpu as pltpu
from jax.experimental.pallas import tpu_sc as plsc

N = 10000
E = 320000
D = 128
K = 64
NCLS = 10
MINN = 1e-15
EPS = 1e-7
MAXNORM = 1.0 - 1e-3

NC = 2
NS = 16
NW = NC * NS
EPW = E // NW
CH = 80
NCHUNK = EPW // CH
NPAD = 10240
RPT = NPAD // NS

BN = 2000


def _artanh(x):
    x = jnp.clip(x, -1.0 + EPS, 1.0 - EPS)
    return 0.5 * jnp.log((1.0 + x) / (1.0 - x))


def _rowsq(x):
    return jnp.sum(x * x, axis=-1, keepdims=True)


def _hyp_in(x):
    n = jnp.maximum(jnp.sqrt(_rowsq(x)), MINN)
    m = jnp.minimum(jnp.tanh(n), MAXNORM)
    return (m / n) * x, m


def _pre_agg(xh, xn, W, b):
    bn = jnp.maximum(jnp.sqrt(_rowsq(b)), MINN)
    bm = jnp.minimum(jnp.tanh(bn), MAXNORM)
    hb = (bm / bn) * b
    y2 = bm * bm

    mx = jnp.dot(xh, W, preferred_element_type=jnp.float32)
    s2 = _rowsq(mx)
    mxn = jnp.maximum(jnp.sqrt(s2), MINN)
    r = jnp.tanh(mxn / xn * _artanh(xn))
    zero = s2 == 0.0
    a = jnp.where(zero, 0.0, jnp.minimum(r, MAXNORM))
    fmv = jnp.where(zero, 0.0, a / mxn)
    mv = fmv * mx
    x2 = a * a
    xy = jnp.sum(mv * hb, axis=-1, keepdims=True)
    A = 1.0 + 2.0 * xy + y2
    B = 1.0 - x2
    den = jnp.maximum(1.0 + 2.0 * xy + x2 * y2, MINN)
    h2 = (A * A * x2 + 2.0 * A * B * xy + B * B * y2) / (den * den)
    hn = jnp.maximum(jnp.sqrt(jnp.maximum(h2, 0.0)), MINN)
    m = jnp.minimum(hn, MAXNORM)
    c0 = _artanh(m) / (hn * den)
    return (c0 * A) * mv + (c0 * B) * hb


def _post_agg(s):
    n = jnp.maximum(jnp.sqrt(_rowsq(s)), MINN)
    m = jnp.minimum(jnp.tanh(n), MAXNORM)
    c1 = _artanh(m) / n
    rs = jax.nn.relu(s)
    n3 = jnp.maximum(c1 * jnp.sqrt(_rowsq(rs)), MINN)
    m3 = jnp.minimum(jnp.tanh(n3), MAXNORM)
    return (m3 / n3 * c1) * rs, m3


def _stage_a_body(x_ref, w_ref, b_ref, xhyp_ref, xt_ref):
    xh, xn = _hyp_in(x_ref[...])
    xhyp_ref[...] = xh
    xt_ref[...] = _pre_agg(xh, xn, w_ref[...], b_ref[...])


def _stage_c_body(p0_ref, p1_ref, w_ref, b_ref, xt_ref):
    out1, m3 = _post_agg(p0_ref[...] + p1_ref[...])
    xt_ref[...] = _pre_agg(out1, m3, w_ref[...], b_ref[...])


def _stage_e_body(p0_ref, p1_ref, xh_ref, c_ref, acc_ref):
    i = pl.program_id(0)
    out2, _ = _post_agg(p0_ref[...] + p1_ref[...])
    out = out2 + xh_ref[...]
    cen = c_ref[...]
    x2 = _rowsq(out)
    y2 = jnp.sum(cen * cen, axis=-1)
    xy = lax.dot_general(out, cen, (((1,), (1,)), ((), ())),
                         preferred_element_type=jnp.float32)
    a = 1.0 - 2.0 * xy + y2[None, :]
    b = 1.0 - x2
    dd = 1.0 - 2.0 * xy + x2 * y2[None, :]
    sq = (a * a * x2 - 2.0 * a * b * xy + b * b * y2[None, :]) / jnp.maximum(dd * dd, MINN)
    nrm = jnp.sqrt(jnp.maximum(sq, 0.0))
    dist = 2.0 * _artanh(nrm)

    @pl.when(i == 0)
    def _():
        acc_ref[...] = jnp.zeros_like(acc_ref)

    acc_ref[...] += jnp.sum(dist, axis=0, keepdims=True)


_row_spec = pl.BlockSpec((BN, D), lambda i: (i, 0))
_whole = lambda shape: pl.BlockSpec(shape, lambda i: tuple(0 for _ in shape))


def _stage_a(x, W1, b1):
    return pl.pallas_call(
        _stage_a_body,
        grid=(N // BN,),
        in_specs=[_row_spec, _whole((D, D)), _whole((1, D))],
        out_specs=[_row_spec, _row_spec],
        out_shape=[jax.ShapeDtypeStruct((N, D), jnp.float32),
                   jax.ShapeDtypeStruct((N, D), jnp.float32)],
    )(x, W1, b1)


def _stage_c(p0, p1, W2, b2):
    return pl.pallas_call(
        _stage_c_body,
        grid=(N // BN,),
        in_specs=[_row_spec, _row_spec, _whole((D, D)), _whole((1, D))],
        out_specs=_row_spec,
        out_shape=jax.ShapeDtypeStruct((N, D), jnp.float32),
    )(p0, p1, W2, b2)


def _stage_e(p0, p1, xhyp, centroids):
    return pl.pallas_call(
        _stage_e_body,
        grid=(N // BN,),
        in_specs=[_row_spec, _row_spec, _row_spec, _whole((K, D))],
        out_specs=_whole((1, K)),
        out_shape=jax.ShapeDtypeStruct((1, K), jnp.float32),
    )(p0, p1, xhyp, centroids)


NI = 5
NR = 3


def _spmm_body(xt_hbm, src_hbm, dst_hbm, w_hbm, zeros_hbm, out_hbm,
               src_v, dst_v, w_v, rows_v, acc_sh, isems, gsems, ssems):
    cid = lax.axis_index("c")
    sid = lax.axis_index("s")
    wid = cid * NS + sid
    base = wid * EPW

    pltpu.sync_copy(zeros_hbm, acc_sh.at[pl.ds(sid * RPT, RPT)])
    plsc.subcore_barrier()

    def i_descs(chunk, xi):
        c = lax.rem(chunk, NCHUNK)
        off = base + c * CH
        return (
            pltpu.make_async_copy(src_hbm.at[pl.ds(off, CH)],
                                  src_v.at[pl.ds(xi * CH, CH)], isems.at[xi]),
            pltpu.make_async_copy(w_hbm.at[pl.ds(off, CH)],
                                  w_v.at[pl.ds(xi * CH, CH)], isems.at[xi]),
            pltpu.make_async_copy(dst_hbm.at[pl.ds(off, CH)], dst_v.at[xi],
                                  isems.at[xi]),
        )

    def i_start(chunk, xi):
        for d in i_descs(chunk, xi):
            d.start()

    def i_wait(chunk, xi):
        for d in i_descs(chunk, xi):
            d.wait()

    def g_desc(r, xi):
        return pltpu.make_async_copy(
            xt_hbm.at[src_v.at[pl.ds(xi * CH, CH)]],
            rows_v.at[pl.ds(r * CH, CH)], gsems.at[r])

    def s_desc(r, xi):
        return pltpu.make_async_copy(rows_v.at[pl.ds(r * CH, CH)],
                                     acc_sh.at[dst_v.at[xi]], ssems.at[r])

    i_start(0, 0)
    i_start(1, 1)
    i_start(2, 2)
    i_wait(0, 0)
    g_desc(0, 0).start()

    @pl.loop(0, NCHUNK)
    def _(k):
        r = lax.rem(k, NR)
        r1 = lax.rem(k + 1, NR)
        xi = lax.rem(k, NI)
        g_desc(r, xi).wait()

        @pl.when(k > 1)
        def _():
            s_desc(r1, lax.rem(k + 3, NI)).wait()

        i_wait(k + 1, lax.rem(k + 1, NI))
        g_desc(r1, lax.rem(k + 1, NI)).start()
        off = r * CH
        woff = xi * CH

        @pl.loop(0, CH, step=16)
        def _(g):
            wv = w_v[pl.ds(woff + g, 16)]
            for j in range(16):
                w = wv[j]
                for q in range(D // 16):
                    sl = (off + g + j, pl.ds(q * 16, 16))
                    rows_v[sl] = rows_v[sl] * w

        s_desc(r, xi).start(add=True)
        i_start(k + 3, lax.rem(k + 3, NI))

    s_desc((NCHUNK - 2) % NR, (NCHUNK - 2) % NI).wait()
    s_desc((NCHUNK - 1) % NR, (NCHUNK - 1) % NI).wait()
    g_desc(NCHUNK % NR, NCHUNK % NI).wait()
    i_wait(0, (NCHUNK + 1) % NI)
    i_wait(0, (NCHUNK + 2) % NI)

    plsc.subcore_barrier()
    pltpu.sync_copy(acc_sh.at[pl.ds(sid * RPT, RPT)],
                    out_hbm.at[cid, pl.ds(sid * RPT, RPT)])


@functools.cache
def _spmm_kernel():
    return pl.kernel(
        _spmm_body,
        out_type=jax.ShapeDtypeStruct((NC, NPAD, D), jnp.float32),
        mesh=plsc.VectorSubcoreMesh(core_axis_name="c", subcore_axis_name="s",
                                    num_cores=NC, num_subcores=NS),
        scratch_types=[
            pltpu.VMEM((NI * CH,), jnp.int32),
            pltpu.VMEM((NI, CH), jnp.int32),
            pltpu.VMEM((NI * CH,), jnp.float32),
            pltpu.VMEM((NR * CH, D), jnp.float32),
            pltpu.VMEM_SHARED((NPAD, D), jnp.float32),
            pltpu.SemaphoreType.DMA((NI,)),
            pltpu.SemaphoreType.DMA((NR,)),
            pltpu.SemaphoreType.DMA((NR,)),
        ],
    )


def _spmm(*args):
    return _spmm_kernel()(*args)


def kernel(x, edge_index, edge_weight, graph_indicator, W1, b1, W2, b2,
           centroids, mlp_W, mlp_b):
    del graph_indicator
    src = edge_index[0]
    dst = edge_index[1]
    b1r = b1.reshape(1, D)
    b2r = b2.reshape(1, D)
    zeros = jnp.zeros((RPT, D), jnp.float32)

    x_hyp, xt1 = _stage_a(x, W1, b1r)
    p1 = _spmm(xt1, src, dst, edge_weight, zeros)
    xt2 = _stage_c(p1[0], p1[1], W2, b2r)
    p2 = _spmm(xt2, src, dst, edge_weight, zeros)
    dist_sum = _stage_e(p2[0], p2[1], x_hyp, centroids)

    readout = dist_sum / jnp.float32(N)
    rn = jnp.maximum(jnp.sqrt(jnp.sum(readout * readout, axis=-1, keepdims=True)), MINN)
    h = _artanh(rn) * readout / rn
    logits = jax.nn.softmax(h @ mlp_W + mlp_b[None, :], axis=0)[None, :, :]
    return (logits, h)


# speedup vs baseline: 8.5164x; 1.0038x over previous
import functools

import jax
import jax.numpy as jnp
from jax import lax
from jax.experimental import pallas as pl
from jax.experimental.pallas import tpu as pltpu
from jax.experimental.pallas import tpu_sc as plsc

N = 10000
E = 320000
D = 128
K = 64
NCLS = 10
MINN = 1e-15
EPS = 1e-7
MAXNORM = 1.0 - 1e-3

NC = 2
NS = 16
NW = NC * NS
EPW = E // NW
CH = 80
NCHUNK = EPW // CH
NPAD = 10240
RPT = NPAD // NS

BN = 1000


def _artanh(x):
    x = jnp.clip(x, -1.0 + EPS, 1.0 - EPS)
    return 0.5 * jnp.log((1.0 + x) / (1.0 - x))


def _rowsq(x):
    return jnp.sum(x * x, axis=-1, keepdims=True)


def _hyp_in(x):
    n = jnp.maximum(jnp.sqrt(_rowsq(x)), MINN)
    m = jnp.minimum(jnp.tanh(n), MAXNORM)
    return (m / n) * x, m


def _pre_agg(xh, xn, W, b):
    bn = jnp.maximum(jnp.sqrt(_rowsq(b)), MINN)
    bm = jnp.minimum(jnp.tanh(bn), MAXNORM)
    hb = (bm / bn) * b
    y2 = bm * bm

    mx = jnp.dot(xh, W, preferred_element_type=jnp.float32)
    s2 = _rowsq(mx)
    mxn = jnp.maximum(jnp.sqrt(s2), MINN)
    r = jnp.tanh(mxn / xn * _artanh(xn))
    zero = s2 == 0.0
    a = jnp.where(zero, 0.0, jnp.minimum(r, MAXNORM))
    fmv = jnp.where(zero, 0.0, a / mxn)
    mv = fmv * mx
    x2 = a * a
    xy = jnp.sum(mv * hb, axis=-1, keepdims=True)
    A = 1.0 + 2.0 * xy + y2
    B = 1.0 - x2
    den = jnp.maximum(1.0 + 2.0 * xy + x2 * y2, MINN)
    h2 = (A * A * x2 + 2.0 * A * B * xy + B * B * y2) / (den * den)
    hn = jnp.maximum(jnp.sqrt(jnp.maximum(h2, 0.0)), MINN)
    m = jnp.minimum(hn, MAXNORM)
    c0 = _artanh(m) / (hn * den)
    return (c0 * A) * mv + (c0 * B) * hb


def _post_agg(s):
    n = jnp.maximum(jnp.sqrt(_rowsq(s)), MINN)
    m = jnp.minimum(jnp.tanh(n), MAXNORM)
    c1 = _artanh(m) / n
    rs = jax.nn.relu(s)
    n3 = jnp.maximum(c1 * jnp.sqrt(_rowsq(rs)), MINN)
    m3 = jnp.minimum(jnp.tanh(n3), MAXNORM)
    return (m3 / n3 * c1) * rs, m3


def _stage_a_body(x_ref, w_ref, b_ref, xhyp_ref, xt_ref):
    xh, xn = _hyp_in(x_ref[...])
    xhyp_ref[...] = xh
    xt_ref[...] = _pre_agg(xh, xn, w_ref[...], b_ref[...])


def _stage_c_body(p0_ref, p1_ref, w_ref, b_ref, xt_ref):
    out1, m3 = _post_agg(p0_ref[...] + p1_ref[...])
    xt_ref[...] = _pre_agg(out1, m3, w_ref[...], b_ref[...])


def _stage_e_body(p0_ref, p1_ref, xh_ref, c_ref, acc_ref):
    i = pl.program_id(0)
    out2, _ = _post_agg(p0_ref[...] + p1_ref[...])
    out = out2 + xh_ref[...]
    cen = c_ref[...]
    x2 = _rowsq(out)
    y2 = jnp.sum(cen * cen, axis=-1)
    xy = lax.dot_general(out, cen, (((1,), (1,)), ((), ())),
                         preferred_element_type=jnp.float32)
    a = 1.0 - 2.0 * xy + y2[None, :]
    b = 1.0 - x2
    dd = 1.0 - 2.0 * xy + x2 * y2[None, :]
    sq = (a * a * x2 - 2.0 * a * b * xy + b * b * y2[None, :]) / jnp.maximum(dd * dd, MINN)
    nrm = jnp.sqrt(jnp.maximum(sq, 0.0))
    dist = 2.0 * _artanh(nrm)

    @pl.when(i == 0)
    def _():
        acc_ref[...] = jnp.zeros_like(acc_ref)

    acc_ref[...] += jnp.sum(dist, axis=0, keepdims=True)


_row_spec = pl.BlockSpec((BN, D), lambda i: (i, 0))
_whole = lambda shape: pl.BlockSpec(shape, lambda i: tuple(0 for _ in shape))


def _stage_a(x, W1, b1):
    return pl.pallas_call(
        _stage_a_body,
        grid=(N // BN,),
        in_specs=[_row_spec, _whole((D, D)), _whole((1, D))],
        out_specs=[_row_spec, _row_spec],
        out_shape=[jax.ShapeDtypeStruct((N, D), jnp.float32),
                   jax.ShapeDtypeStruct((N, D), jnp.float32)],
    )(x, W1, b1)


def _stage_c(p0, p1, W2, b2):
    return pl.pallas_call(
        _stage_c_body,
        grid=(N // BN,),
        in_specs=[_row_spec, _row_spec, _whole((D, D)), _whole((1, D))],
        out_specs=_row_spec,
        out_shape=jax.ShapeDtypeStruct((N, D), jnp.float32),
    )(p0, p1, W2, b2)


def _stage_e(p0, p1, xhyp, centroids):
    return pl.pallas_call(
        _stage_e_body,
        grid=(N // BN,),
        in_specs=[_row_spec, _row_spec, _row_spec, _whole((K, D))],
        out_specs=_whole((1, K)),
        out_shape=jax.ShapeDtypeStruct((1, K), jnp.float32),
    )(p0, p1, xhyp, centroids)


NI = 5
NR = 3


def _spmm_body(xt_hbm, src_hbm, dst_hbm, w_hbm, zeros_hbm, out_hbm,
               src_v, dst_v, w_v, rows_v, acc_sh, isems, gsems, ssems):
    cid = lax.axis_index("c")
    sid = lax.axis_index("s")
    wid = cid * NS + sid
    base = wid * EPW

    pltpu.sync_copy(zeros_hbm, acc_sh.at[pl.ds(sid * RPT, RPT)])
    plsc.subcore_barrier()

    def i_descs(chunk, xi):
        c = lax.rem(chunk, NCHUNK)
        off = base + c * CH
        return (
            pltpu.make_async_copy(src_hbm.at[pl.ds(off, CH)],
                                  src_v.at[pl.ds(xi * CH, CH)], isems.at[xi]),
            pltpu.make_async_copy(w_hbm.at[pl.ds(off, CH)],
                                  w_v.at[pl.ds(xi * CH, CH)], isems.at[xi]),
            pltpu.make_async_copy(dst_hbm.at[pl.ds(off, CH)], dst_v.at[xi],
                                  isems.at[xi]),
        )

    def i_start(chunk, xi):
        for d in i_descs(chunk, xi):
            d.start()

    def i_wait(chunk, xi):
        for d in i_descs(chunk, xi):
            d.wait()

    def g_desc(r, xi):
        return pltpu.make_async_copy(
            xt_hbm.at[src_v.at[pl.ds(xi * CH, CH)]],
            rows_v.at[pl.ds(r * CH, CH)], gsems.at[r])

    def s_desc(r, xi):
        return pltpu.make_async_copy(rows_v.at[pl.ds(r * CH, CH)],
                                     acc_sh.at[dst_v.at[xi]], ssems.at[r])

    i_start(0, 0)
    i_start(1, 1)
    i_start(2, 2)
    i_wait(0, 0)
    g_desc(0, 0).start()

    @pl.loop(0, NCHUNK)
    def _(k):
        r = lax.rem(k, NR)
        r1 = lax.rem(k + 1, NR)
        xi = lax.rem(k, NI)
        g_desc(r, xi).wait()

        @pl.when(k > 1)
        def _():
            s_desc(r1, lax.rem(k + 3, NI)).wait()

        i_wait(k + 1, lax.rem(k + 1, NI))
        g_desc(r1, lax.rem(k + 1, NI)).start()
        off = r * CH
        woff = xi * CH

        @pl.loop(0, CH, step=16)
        def _(g):
            wv = w_v[pl.ds(woff + g, 16)]
            for j in range(16):
                w = wv[j]
                for q in range(D // 16):
                    sl = (off + g + j, pl.ds(q * 16, 16))
                    rows_v[sl] = rows_v[sl] * w

        s_desc(r, xi).start(add=True)
        i_start(k + 3, lax.rem(k + 3, NI))

    s_desc((NCHUNK - 2) % NR, (NCHUNK - 2) % NI).wait()
    s_desc((NCHUNK - 1) % NR, (NCHUNK - 1) % NI).wait()
    g_desc(NCHUNK % NR, NCHUNK % NI).wait()
    i_wait(0, (NCHUNK + 1) % NI)
    i_wait(0, (NCHUNK + 2) % NI)

    plsc.subcore_barrier()
    pltpu.sync_copy(acc_sh.at[pl.ds(sid * RPT, RPT)],
                    out_hbm.at[cid, pl.ds(sid * RPT, RPT)])


@functools.cache
def _spmm_kernel():
    return pl.kernel(
        _spmm_body,
        out_type=jax.ShapeDtypeStruct((NC, NPAD, D), jnp.float32),
        mesh=plsc.VectorSubcoreMesh(core_axis_name="c", subcore_axis_name="s",
                                    num_cores=NC, num_subcores=NS),
        scratch_types=[
            pltpu.VMEM((NI * CH,), jnp.int32),
            pltpu.VMEM((NI, CH), jnp.int32),
            pltpu.VMEM((NI * CH,), jnp.float32),
            pltpu.VMEM((NR * CH, D), jnp.float32),
            pltpu.VMEM_SHARED((NPAD, D), jnp.float32),
            pltpu.SemaphoreType.DMA((NI,)),
            pltpu.SemaphoreType.DMA((NR,)),
            pltpu.SemaphoreType.DMA((NR,)),
        ],
    )


def _spmm(*args):
    return _spmm_kernel()(*args)


def kernel(x, edge_index, edge_weight, graph_indicator, W1, b1, W2, b2,
           centroids, mlp_W, mlp_b):
    del graph_indicator
    src = edge_index[0]
    dst = edge_index[1]
    b1r = b1.reshape(1, D)
    b2r = b2.reshape(1, D)
    zeros = jnp.zeros((RPT, D), jnp.float32)

    x_hyp, xt1 = _stage_a(x, W1, b1r)
    p1 = _spmm(xt1, src, dst, edge_weight, zeros)
    xt2 = _stage_c(p1[0], p1[1], W2, b2r)
    p2 = _spmm(xt2, src, dst, edge_weight, zeros)
    dist_sum = _stage_e(p2[0], p2[1], x_hyp, centroids)

    readout = dist_sum / jnp.float32(N)
    rn = jnp.maximum(jnp.sqrt(jnp.sum(readout * readout, axis=-1, keepdims=True)), MINN)
    h = _artanh(rn) * readout / rn
    logits = jax.nn.softmax(h @ mlp_W + mlp_b[None, :], axis=0)[None, :, :]
    return (logits, h)
